# baseline (speedup 1.0000x reference)
import functools

import jax
import jax.numpy as jnp
from jax.experimental import pallas as pl
from jax.experimental.pallas import tpu as pltpu
from jax.experimental.pallas import tpu_sc as plsc


def _router_body(x_ref, wg_ref, logits_ref, comb_ref):
    x = x_ref[...]
    wg = wg_ref[...]
    logits = jax.lax.dot_general(
        x, wg, (((1,), (1,)), ((), ())),
        preferred_element_type=jnp.float32,
    )
    logits_ref[...] = logits
    m = jnp.max(logits, axis=1, keepdims=True)
    ex = jnp.exp(logits - m)
    p = ex / jnp.sum(ex, axis=1, keepdims=True)
    T, E = p.shape
    eidx = jax.lax.broadcasted_iota(jnp.int32, (T, E), 1)
    a1 = jnp.argmax(p, axis=1).astype(jnp.int32)
    m1 = jnp.max(p, axis=1)
    mask1 = eidx == a1[:, None]
    p2 = jnp.where(mask1, -1.0, p)
    a2 = jnp.argmax(p2, axis=1).astype(jnp.int32)
    m2 = jnp.max(p2, axis=1)
    denom = m1 + m2
    w1 = m1 / denom
    w2 = m2 / denom
    mask2 = eidx == a2[:, None]
    comb_ref[...] = (jnp.where(mask1, w1[:, None], 0.0)
                     + jnp.where(mask2, w2[:, None], 0.0))


def _gemm_body(be_ref, xs_ref, wg_ref, wu_ref, wd_ref, ys_ref):
    xb = xs_ref[...]
    wg = wg_ref[0]
    wu = wu_ref[0]
    wd = wd_ref[0]
    g = jax.lax.dot_general(xb, wg, (((1,), (1,)), ((), ())),
                            preferred_element_type=jnp.float32)
    u = jax.lax.dot_general(xb, wu, (((1,), (1,)), ((), ())),
                            preferred_element_type=jnp.float32)
    h = (g * jax.lax.logistic(g) * u).astype(jnp.bfloat16)
    y = jax.lax.dot_general(h, wd, (((1,), (1,)), ((), ())),
                            preferred_element_type=jnp.float32)
    ys_ref[...] = y.astype(jnp.bfloat16)


def _combine_body(y12_ref, w1_ref, w2_ref, out_ref, *, d):
    y = y12_ref[...].astype(jnp.float32)
    out_ref[...] = y[:, :d] * w1_ref[...] + y[:, d:] * w2_ref[...]


def _sc_gather_bf16(data, idx):
    n_rows, c = data.shape
    sub = c // 256
    d32 = jax.lax.bitcast_convert_type(
        data.reshape(n_rows * sub, 128, 2), jnp.int32)
    idx_s = (idx[:, None] * sub
             + jnp.arange(sub, dtype=jnp.int32)[None, :]).reshape(-1)
    out32 = _sc_gather(d32, idx_s, 128)
    out = jax.lax.bitcast_convert_type(out32, jnp.bfloat16)
    return out.reshape(idx.shape[0], c)


def _sc_gather(data, idx, window):
    n = idx.shape[0]
    dcols = data.shape[1]
    idx2 = idx.reshape(1, n)
    mesh = plsc.VectorSubcoreMesh(core_axis_name="core",
                                  subcore_axis_name="subcore")

    @functools.partial(
        pl.kernel,
        out_type=jax.ShapeDtypeStruct((n, dcols), data.dtype),
        mesh=mesh)
    def gather_kernel(x_hbm, i_hbm, o_hbm):
        def body(i_vmem, o_vmem):
            pltpu.sync_copy(x_hbm.at[i_vmem.at[0]], o_vmem)

        pltpu.emit_pipeline(
            body,
            grid=(n // window,),
            in_specs=[pl.BlockSpec((1, window), index_map=lambda i: (0, i))],
            out_specs=[pl.BlockSpec((window, dcols),
                                    index_map=lambda i: (i, 0))],
            core_axis_name=("core", "subcore"),
            dimension_semantics=(pltpu.PARALLEL,),
        )(i_hbm, o_hbm)

    return gather_kernel(data, idx2)


def kernel(hidden_states, W_gate, W_g, W_u, W_d):
    b, s, d = hidden_states.shape
    x = hidden_states.reshape(-1, d)
    T, D = x.shape
    E, F, _ = W_g.shape
    K = 2
    BLK = 256
    NB = T * K // BLK + E - 1
    P = NB * BLK

    logits, comb = pl.pallas_call(
        _router_body,
        out_shape=(
            jax.ShapeDtypeStruct((T, E), jnp.float32),
            jax.ShapeDtypeStruct((T, E), jnp.float32),
        ),
    )(x, W_gate)

    eidx = jnp.arange(E, dtype=jnp.int32)[None, :]
    w1 = jnp.max(comb, axis=1)
    e1 = jnp.argmax(comb, axis=1).astype(jnp.int32)
    comb2 = jnp.where(eidx == e1[:, None], -1.0, comb)
    w2 = jnp.max(comb2, axis=1)
    e2 = jnp.argmax(comb2, axis=1).astype(jnp.int32)

    flat_e = jnp.stack([e1, e2], axis=1).reshape(-1)
    oh = (flat_e[:, None] == eidx).astype(jnp.int32)
    csum = jnp.cumsum(oh, axis=0)
    rank = jnp.take_along_axis(csum, flat_e[:, None], axis=1)[:, 0] - 1
    counts = csum[-1]
    nblk = (counts + BLK - 1) // BLK
    seg_start_blk = jnp.concatenate(
        [jnp.zeros((1,), jnp.int32), jnp.cumsum(nblk)[:-1].astype(jnp.int32)])
    dest = seg_start_blk[flat_e] * BLK + rank
    tok_ids = jnp.arange(T * K, dtype=jnp.int32) // K
    tok_idx = jnp.zeros((P,), jnp.int32).at[dest].set(tok_ids)
    seg_end_blk = seg_start_blk + nblk
    bidx = jnp.arange(NB, dtype=jnp.int32)
    blk_expert = jnp.minimum(
        jnp.sum((bidx[:, None] >= seg_end_blk[None, :]).astype(jnp.int32),
                axis=1), E - 1).astype(jnp.int32)

    x_bf = x.astype(jnp.bfloat16)
    xs = _sc_gather_bf16(x_bf, tok_idx)

    wg_bf = W_g.astype(jnp.bfloat16)
    wu_bf = W_u.astype(jnp.bfloat16)
    wd_bf = W_d.astype(jnp.bfloat16)
    grid_spec = pltpu.PrefetchScalarGridSpec(
        num_scalar_prefetch=1,
        grid=(NB,),
        in_specs=[
            pl.BlockSpec((BLK, D), lambda i, be: (i, 0)),
            pl.BlockSpec((1, F, D), lambda i, be: (be[i], 0, 0)),
            pl.BlockSpec((1, F, D), lambda i, be: (be[i], 0, 0)),
            pl.BlockSpec((1, D, F), lambda i, be: (be[i], 0, 0)),
        ],
        out_specs=pl.BlockSpec((BLK, D), lambda i, be: (i, 0)),
    )
    ys = pl.pallas_call(
        _gemm_body,
        grid_spec=grid_spec,
        out_shape=jax.ShapeDtypeStruct((P, D), jnp.bfloat16),
    )(blk_expert, xs, wg_bf, wu_bf, wd_bf)

    y12 = _sc_gather_bf16(ys, dest)
    y12 = y12.reshape(T, K * D)

    BT = 256
    out = pl.pallas_call(
        functools.partial(_combine_body, d=D),
        grid=(T // BT,),
        in_specs=[
            pl.BlockSpec((BT, K * D), lambda t: (t, 0)),
            pl.BlockSpec((BT, 1), lambda t: (t, 0)),
            pl.BlockSpec((BT, 1), lambda t: (t, 0)),
        ],
        out_specs=pl.BlockSpec((BT, D), lambda t: (t, 0)),
        out_shape=jax.ShapeDtypeStruct((T, D), jnp.float32),
    )(y12, w1[:, None], w2[:, None])

    return out.reshape(b, s, d), logits.reshape(b, s, E)


# speedup vs baseline: 23.2714x; 23.2714x over previous
import functools

import jax
import jax.numpy as jnp
from jax.experimental import pallas as pl
from jax.experimental.pallas import tpu as pltpu
from jax.experimental.pallas import tpu_sc as plsc


def _router_body(x_ref, wg_ref, logits_ref, comb_ref):
    x = x_ref[...]
    wg = wg_ref[...]
    logits = jax.lax.dot_general(
        x, wg, (((1,), (1,)), ((), ())),
        preferred_element_type=jnp.float32,
    )
    logits_ref[...] = logits
    m = jnp.max(logits, axis=1, keepdims=True)
    ex = jnp.exp(logits - m)
    p = ex / jnp.sum(ex, axis=1, keepdims=True)
    T, E = p.shape
    eidx = jax.lax.broadcasted_iota(jnp.int32, (T, E), 1)
    a1 = jnp.argmax(p, axis=1).astype(jnp.int32)
    m1 = jnp.max(p, axis=1)
    mask1 = eidx == a1[:, None]
    p2 = jnp.where(mask1, -1.0, p)
    a2 = jnp.argmax(p2, axis=1).astype(jnp.int32)
    m2 = jnp.max(p2, axis=1)
    denom = m1 + m2
    w1 = m1 / denom
    w2 = m2 / denom
    mask2 = eidx == a2[:, None]
    comb_ref[...] = (jnp.where(mask1, w1[:, None], 0.0)
                     + jnp.where(mask2, w2[:, None], 0.0))


def _gemm_body(be_ref, xs_ref, wg_ref, wu_ref, wd_ref, ys_ref):
    xb = xs_ref[...]
    wg = wg_ref[0]
    wu = wu_ref[0]
    wd = wd_ref[0]
    g = jax.lax.dot_general(xb, wg, (((1,), (1,)), ((), ())),
                            preferred_element_type=jnp.float32)
    u = jax.lax.dot_general(xb, wu, (((1,), (1,)), ((), ())),
                            preferred_element_type=jnp.float32)
    h = (g * jax.lax.logistic(g) * u).astype(jnp.bfloat16)
    y = jax.lax.dot_general(h, wd, (((1,), (1,)), ((), ())),
                            preferred_element_type=jnp.float32)
    ys_ref[...] = y.astype(jnp.bfloat16)


def _combine_body(y12_ref, w1_ref, w2_ref, out_ref, *, d):
    y = y12_ref[...].astype(jnp.float32)
    out_ref[...] = y[:, :d] * w1_ref[...] + y[:, d:] * w2_ref[...]


def _sc_gather_bf16(data, idx):
    n_rows, c = data.shape
    sub = c // 256
    d32 = jax.lax.bitcast_convert_type(
        data.reshape(n_rows * sub, 128, 2), jnp.int32)
    idx_s = (idx[:, None] * sub
             + jnp.arange(sub, dtype=jnp.int32)[None, :]).reshape(-1)
    out32 = _sc_gather(d32, idx_s, 128)
    out = jax.lax.bitcast_convert_type(out32, jnp.bfloat16)
    return out.reshape(idx.shape[0], c)


def _sc_gather(data, idx, window):
    n = idx.shape[0]
    dcols = data.shape[1]
    idx2 = idx.reshape(1, n)
    mesh = plsc.VectorSubcoreMesh(core_axis_name="core",
                                  subcore_axis_name="subcore")

    @functools.partial(
        pl.kernel,
        out_type=jax.ShapeDtypeStruct((n, dcols), data.dtype),
        mesh=mesh)
    def gather_kernel(x_hbm, i_hbm, o_hbm):
        def body(i_vmem, o_vmem):
            pltpu.sync_copy(x_hbm.at[i_vmem.at[0]], o_vmem)

        pltpu.emit_pipeline(
            body,
            grid=(n // window,),
            in_specs=[pl.BlockSpec((1, window), index_map=lambda i: (0, i))],
            out_specs=[pl.BlockSpec((window, dcols),
                                    index_map=lambda i: (i, 0))],
            core_axis_name=("core", "subcore"),
            dimension_semantics=(pltpu.PARALLEL,),
        )(i_hbm, o_hbm)

    return gather_kernel(data, idx2)


def kernel(hidden_states, W_gate, W_g, W_u, W_d):
    b, s, d = hidden_states.shape
    x = hidden_states.reshape(-1, d)
    T, D = x.shape
    E, F, _ = W_g.shape
    K = 2
    BLK = 256
    NB = T * K // BLK + E - 1
    P = NB * BLK

    logits, comb = pl.pallas_call(
        _router_body,
        out_shape=(
            jax.ShapeDtypeStruct((T, E), jnp.float32),
            jax.ShapeDtypeStruct((T, E), jnp.float32),
        ),
    )(x, W_gate)

    eidx = jnp.arange(E, dtype=jnp.int32)[None, :]
    w1 = jnp.max(comb, axis=1)
    e1 = jnp.argmax(comb, axis=1).astype(jnp.int32)
    comb2 = jnp.where(eidx == e1[:, None], -1.0, comb)
    w2 = jnp.max(comb2, axis=1)
    e2 = jnp.argmax(comb2, axis=1).astype(jnp.int32)

    flat_e = jnp.stack([e1, e2], axis=1).reshape(-1)
    oh = (flat_e[:, None] == eidx).astype(jnp.int32)
    csum = jnp.cumsum(oh, axis=0)
    rank = jnp.take_along_axis(csum, flat_e[:, None], axis=1)[:, 0] - 1
    counts = csum[-1]
    nblk = (counts + BLK - 1) // BLK
    seg_start_blk = jnp.concatenate(
        [jnp.zeros((1,), jnp.int32), jnp.cumsum(nblk)[:-1].astype(jnp.int32)])
    dest = seg_start_blk[flat_e] * BLK + rank
    tok_ids = jnp.arange(T * K, dtype=jnp.int32) // K
    tok_idx = jnp.zeros((P,), jnp.int32).at[dest].set(tok_ids)
    seg_end_blk = seg_start_blk + nblk
    bidx = jnp.arange(NB, dtype=jnp.int32)
    blk_expert = jnp.minimum(
        jnp.sum((bidx[:, None] >= seg_end_blk[None, :]).astype(jnp.int32),
                axis=1), E - 1).astype(jnp.int32)

    x_bf = x.astype(jnp.bfloat16)
    xs = jnp.take(x_bf, tok_idx, axis=0)

    wg_bf = W_g.astype(jnp.bfloat16)
    wu_bf = W_u.astype(jnp.bfloat16)
    wd_bf = W_d.astype(jnp.bfloat16)
    grid_spec = pltpu.PrefetchScalarGridSpec(
        num_scalar_prefetch=1,
        grid=(NB,),
        in_specs=[
            pl.BlockSpec((BLK, D), lambda i, be: (i, 0)),
            pl.BlockSpec((1, F, D), lambda i, be: (be[i], 0, 0)),
            pl.BlockSpec((1, F, D), lambda i, be: (be[i], 0, 0)),
            pl.BlockSpec((1, D, F), lambda i, be: (be[i], 0, 0)),
        ],
        out_specs=pl.BlockSpec((BLK, D), lambda i, be: (i, 0)),
    )
    ys = pl.pallas_call(
        _gemm_body,
        grid_spec=grid_spec,
        out_shape=jax.ShapeDtypeStruct((P, D), jnp.bfloat16),
    )(blk_expert, xs, wg_bf, wu_bf, wd_bf)

    y12 = jnp.take(ys, dest, axis=0)
    y12 = y12.reshape(T, K * D)

    BT = 256
    out = pl.pallas_call(
        functools.partial(_combine_body, d=D),
        grid=(T // BT,),
        in_specs=[
            pl.BlockSpec((BT, K * D), lambda t: (t, 0)),
            pl.BlockSpec((BT, 1), lambda t: (t, 0)),
            pl.BlockSpec((BT, 1), lambda t: (t, 0)),
        ],
        out_specs=pl.BlockSpec((BT, D), lambda t: (t, 0)),
        out_shape=jax.ShapeDtypeStruct((T, D), jnp.float32),
    )(y12, w1[:, None], w2[:, None])

    return out.reshape(b, s, d), logits.reshape(b, s, E)


# speedup vs baseline: 35.2255x; 1.5137x over previous
import functools

import jax
import jax.numpy as jnp
from jax import lax
from jax.experimental import pallas as pl
from jax.experimental.pallas import tpu as pltpu
from jax.experimental.pallas import tpu_sc as plsc


def _router_body(x_ref, wg_ref, logits_ref, comb_ref, xpack_ref):
    x = x_ref[...]
    wg = wg_ref[...]
    logits = jax.lax.dot_general(
        x, wg, (((1,), (1,)), ((), ())),
        preferred_element_type=jnp.float32,
    )
    logits_ref[...] = logits
    T, E = logits.shape
    m = jnp.max(logits, axis=1, keepdims=True)
    ex = jnp.exp(logits - m)
    p = ex / jnp.sum(ex, axis=1, keepdims=True)
    eidx = jax.lax.broadcasted_iota(jnp.int32, (T, E), 1)
    a1 = jnp.argmax(p, axis=1).astype(jnp.int32)
    m1 = jnp.max(p, axis=1)
    mask1 = eidx == a1[:, None]
    p2 = jnp.where(mask1, -1.0, p)
    a2 = jnp.argmax(p2, axis=1).astype(jnp.int32)
    m2 = jnp.max(p2, axis=1)
    denom = m1 + m2
    w1 = m1 / denom
    w2 = m2 / denom
    mask2 = eidx == a2[:, None]
    comb_ref[...] = (jnp.where(mask1, w1[:, None], 0.0)
                     + jnp.where(mask2, w2[:, None], 0.0))
    xpack_ref[...] = x.reshape(xpack_ref.shape)


def _sc_scatter_body(x_hbm, idx_hbm, o_hbm, idx_v, rows_v, sem, *, chunks):
    nc = 2
    bpw = idx_v.shape[0]
    wid = lax.axis_index("subcore") * nc + lax.axis_index("core")
    t = x_hbm.shape[0]
    for c in range(chunks):
        base = wid * (bpw * chunks) + c * bpw
        src = lax.rem(base, t)
        pltpu.sync_copy(idx_hbm.at[pl.ds(base, bpw)], idx_v)
        pltpu.sync_copy(x_hbm.at[pl.ds(src, bpw)], rows_v)
        pltpu.async_copy(rows_v, o_hbm.at[idx_v], sem).wait()


def _sc_gather_body(ys_hbm, idx_hbm, o_hbm, idx_v, rows_v, sem, *, chunks):
    nc = 2
    bpw = idx_v.shape[0]
    wid = lax.axis_index("subcore") * nc + lax.axis_index("core")
    for c in range(chunks):
        base = wid * (bpw * chunks) + c * bpw
        pltpu.sync_copy(idx_hbm.at[pl.ds(base, bpw)], idx_v)
        pltpu.async_copy(ys_hbm.at[idx_v], rows_v, sem).wait()
        pltpu.sync_copy(rows_v, o_hbm.at[pl.ds(base, bpw)])


_SC_MESH = dict(core_axis_name="core", subcore_axis_name="subcore")
_NW = 32


_CHUNKS = 2


def _sc_scatter_rows(x_pack, dest, p):
    a = dest.shape[0]
    _, sl, lanes = x_pack.shape
    bpw = a // (_NW * _CHUNKS)
    run = pl.kernel(
        functools.partial(_sc_scatter_body, chunks=_CHUNKS),
        out_type=jax.ShapeDtypeStruct((p, sl, lanes), x_pack.dtype),
        mesh=plsc.VectorSubcoreMesh(**_SC_MESH),
        scratch_types=[
            pltpu.VMEM((bpw,), jnp.int32),
            pltpu.VMEM((bpw, sl, lanes), x_pack.dtype),
            pltpu.SemaphoreType.DMA,
        ],
    )
    return run(x_pack, dest)


def _sc_gather_rows(ys, dest):
    a = dest.shape[0]
    _, sl, lanes = ys.shape
    bpw = a // (_NW * _CHUNKS)
    run = pl.kernel(
        functools.partial(_sc_gather_body, chunks=_CHUNKS),
        out_type=jax.ShapeDtypeStruct((a, sl, lanes), ys.dtype),
        mesh=plsc.VectorSubcoreMesh(**_SC_MESH),
        scratch_types=[
            pltpu.VMEM((bpw,), jnp.int32),
            pltpu.VMEM((bpw, sl, lanes), ys.dtype),
            pltpu.SemaphoreType.DMA,
        ],
    )
    return run(ys, dest)


def _gemm_body(be_ref, xs_ref, wg_ref, wu_ref, wd_ref, ys_ref):
    blk = xs_ref.shape[0]
    d = wg_ref.shape[2] * 1
    xb = xs_ref[...].reshape(blk, -1).astype(jnp.bfloat16)
    wg = wg_ref[0]
    wu = wu_ref[0]
    wd = wd_ref[0]
    g = jax.lax.dot_general(xb, wg, (((1,), (1,)), ((), ())),
                            preferred_element_type=jnp.float32)
    u = jax.lax.dot_general(xb, wu, (((1,), (1,)), ((), ())),
                            preferred_element_type=jnp.float32)
    h = (g * jax.lax.logistic(g) * u).astype(jnp.bfloat16)
    y = jax.lax.dot_general(h, wd, (((1,), (1,)), ((), ())),
                            preferred_element_type=jnp.float32)
    ys_ref[...] = y.reshape(ys_ref.shape)


def _combine_body(y1_ref, y2_ref, w1_ref, w2_ref, out_ref):
    bt = y1_ref.shape[1]
    y1 = y1_ref[0].reshape(bt, -1).astype(jnp.float32)
    y2 = y2_ref[0].reshape(bt, -1).astype(jnp.float32)
    out_ref[...] = y1 * w1_ref[...] + y2 * w2_ref[...]


def kernel(hidden_states, W_gate, W_g, W_u, W_d):
    b, s, d = hidden_states.shape
    x = hidden_states.reshape(-1, d)
    T, D = x.shape
    E, F, _ = W_g.shape
    K = 2
    BLK = 256
    NB = T * K // BLK + E - 1
    P = NB * BLK
    SL = D // 128
    A = T * K
    NW = 32
    BPW = A // NW

    logits, comb, x_pack = pl.pallas_call(
        _router_body,
        out_shape=(
            jax.ShapeDtypeStruct((T, E), jnp.float32),
            jax.ShapeDtypeStruct((T, E), jnp.float32),
            jax.ShapeDtypeStruct((T, SL, 128), jnp.float32),
        ),
    )(x, W_gate)

    eidx = jnp.arange(E, dtype=jnp.int32)[None, :]
    w1 = jnp.max(comb, axis=1)
    e1 = jnp.argmax(comb, axis=1).astype(jnp.int32)
    comb2 = jnp.where(eidx == e1[:, None], -1.0, comb)
    w2 = jnp.max(comb2, axis=1)
    e2 = jnp.argmax(comb2, axis=1).astype(jnp.int32)

    flat_e = jnp.concatenate([e1, e2])
    oh = (flat_e[:, None] == eidx).astype(jnp.int32)
    csum = jnp.cumsum(oh, axis=0)
    rank = jnp.take_along_axis(csum, flat_e[:, None], axis=1)[:, 0] - 1
    counts = csum[-1]
    nblk = (counts + BLK - 1) // BLK
    seg_start_blk = jnp.concatenate(
        [jnp.zeros((1,), jnp.int32), jnp.cumsum(nblk)[:-1].astype(jnp.int32)])
    dest = seg_start_blk[flat_e] * BLK + rank
    seg_end_blk = seg_start_blk + nblk
    bidx = jnp.arange(NB, dtype=jnp.int32)
    blk_expert = jnp.minimum(
        jnp.sum((bidx[:, None] >= seg_end_blk[None, :]).astype(jnp.int32),
                axis=1), E - 1).astype(jnp.int32)

    xs = _sc_scatter_rows(x_pack, dest, P)

    wg_bf = W_g.astype(jnp.bfloat16)
    wu_bf = W_u.astype(jnp.bfloat16)
    wd_bf = W_d.astype(jnp.bfloat16)
    grid_spec = pltpu.PrefetchScalarGridSpec(
        num_scalar_prefetch=1,
        grid=(NB,),
        in_specs=[
            pl.BlockSpec((BLK, SL, 128), lambda i, be: (i, 0, 0)),
            pl.BlockSpec((1, F, D), lambda i, be: (be[i], 0, 0)),
            pl.BlockSpec((1, F, D), lambda i, be: (be[i], 0, 0)),
            pl.BlockSpec((1, D, F), lambda i, be: (be[i], 0, 0)),
        ],
        out_specs=pl.BlockSpec((BLK, SL, 128), lambda i, be: (i, 0, 0)),
    )
    ys = pl.pallas_call(
        _gemm_body,
        grid_spec=grid_spec,
        out_shape=jax.ShapeDtypeStruct((P, SL, 128), jnp.float32),
    )(blk_expert, xs, wg_bf, wu_bf, wd_bf)

    y12 = _sc_gather_rows(ys, dest)
    y12 = y12.reshape(K, T, SL, 128)

    BT = 256
    out = pl.pallas_call(
        _combine_body,
        grid=(T // BT,),
        in_specs=[
            pl.BlockSpec((1, BT, SL, 128), lambda t: (0, t, 0, 0)),
            pl.BlockSpec((1, BT, SL, 128), lambda t: (1, t, 0, 0)),
            pl.BlockSpec((BT, 1), lambda t: (t, 0)),
            pl.BlockSpec((BT, 1), lambda t: (t, 0)),
        ],
        out_specs=pl.BlockSpec((BT, D), lambda t: (t, 0)),
        out_shape=jax.ShapeDtypeStruct((T, D), jnp.float32),
    )(y12, y12, w1[:, None], w2[:, None])

    return out.reshape(b, s, d), logits.reshape(b, s, E)


# speedup vs baseline: 42.3713x; 1.2029x over previous
import functools

import jax
import jax.numpy as jnp
from jax import lax
from jax.experimental import pallas as pl
from jax.experimental.pallas import tpu as pltpu
from jax.experimental.pallas import tpu_sc as plsc


def _router_body(x_ref, wg_ref, logits_ref, comb_ref, xpack_ref):
    x = x_ref[...]
    wg = wg_ref[...]
    logits = jax.lax.dot_general(
        x, wg, (((1,), (1,)), ((), ())),
        preferred_element_type=jnp.float32,
    )
    logits_ref[...] = logits
    T, E = logits.shape
    m = jnp.max(logits, axis=1, keepdims=True)
    ex = jnp.exp(logits - m)
    p = ex / jnp.sum(ex, axis=1, keepdims=True)
    eidx = jax.lax.broadcasted_iota(jnp.int32, (T, E), 1)
    a1 = jnp.argmax(p, axis=1).astype(jnp.int32)
    m1 = jnp.max(p, axis=1)
    mask1 = eidx == a1[:, None]
    p2 = jnp.where(mask1, -1.0, p)
    a2 = jnp.argmax(p2, axis=1).astype(jnp.int32)
    m2 = jnp.max(p2, axis=1)
    denom = m1 + m2
    w1 = m1 / denom
    w2 = m2 / denom
    mask2 = eidx == a2[:, None]
    comb_ref[...] = (jnp.where(mask1, w1[:, None], 0.0)
                     + jnp.where(mask2, w2[:, None], 0.0))
    xpack_ref[...] = x.reshape(xpack_ref.shape)


def _sc_scatter_body(x_hbm, idx_hbm, o_hbm, idx_v, rows_v, sem, *, chunks):
    nc = 2
    bpw = idx_v.shape[0]
    wid = lax.axis_index("subcore") * nc + lax.axis_index("core")
    t = x_hbm.shape[0]
    for c in range(chunks):
        base = wid * (bpw * chunks) + c * bpw
        src = lax.rem(base, t)
        pltpu.sync_copy(idx_hbm.at[pl.ds(base, bpw)], idx_v)
        pltpu.sync_copy(x_hbm.at[pl.ds(src, bpw)], rows_v)
        pltpu.async_copy(rows_v, o_hbm.at[idx_v], sem).wait()


def _sc_gather_body(ys_hbm, idx_hbm, o_hbm, idx_v, rows_v, sem, *, chunks):
    nc = 2
    bpw = idx_v.shape[0]
    wid = lax.axis_index("subcore") * nc + lax.axis_index("core")
    for c in range(chunks):
        base = wid * (bpw * chunks) + c * bpw
        pltpu.sync_copy(idx_hbm.at[pl.ds(base, bpw)], idx_v)
        pltpu.async_copy(ys_hbm.at[idx_v], rows_v, sem).wait()
        pltpu.sync_copy(rows_v, o_hbm.at[pl.ds(base, bpw)])


_SC_MESH = dict(core_axis_name="core", subcore_axis_name="subcore")
_NW = 32


_CHUNKS = 2


def _sc_scatter_rows(x_pack, dest, p):
    a = dest.shape[0]
    _, sl, lanes = x_pack.shape
    bpw = a // (_NW * _CHUNKS)
    run = pl.kernel(
        functools.partial(_sc_scatter_body, chunks=_CHUNKS),
        out_type=jax.ShapeDtypeStruct((p, sl, lanes), x_pack.dtype),
        mesh=plsc.VectorSubcoreMesh(**_SC_MESH),
        scratch_types=[
            pltpu.VMEM((bpw,), jnp.int32),
            pltpu.VMEM((bpw, sl, lanes), x_pack.dtype),
            pltpu.SemaphoreType.DMA,
        ],
    )
    return run(x_pack, dest)


def _sc_gather_rows(ys, dest):
    a = dest.shape[0]
    _, sl, lanes = ys.shape
    bpw = a // (_NW * _CHUNKS)
    run = pl.kernel(
        functools.partial(_sc_gather_body, chunks=_CHUNKS),
        out_type=jax.ShapeDtypeStruct((a, sl, lanes), ys.dtype),
        mesh=plsc.VectorSubcoreMesh(**_SC_MESH),
        scratch_types=[
            pltpu.VMEM((bpw,), jnp.int32),
            pltpu.VMEM((bpw, sl, lanes), ys.dtype),
            pltpu.SemaphoreType.DMA,
        ],
    )
    return run(ys, dest)


def _gemm_body(be_ref, xs_ref, wg_ref, wu_ref, wd_ref, ys_ref):
    nb = pl.num_programs(0)

    @pl.when(pl.program_id(0) < be_ref[nb])
    def _compute():
        blk = xs_ref.shape[0]
        xb = xs_ref[...].reshape(blk, -1).astype(jnp.bfloat16)
        wg = wg_ref[0].astype(jnp.bfloat16)
        wu = wu_ref[0].astype(jnp.bfloat16)
        wd = wd_ref[0].astype(jnp.bfloat16)
        g = jax.lax.dot_general(xb, wg, (((1,), (1,)), ((), ())),
                                preferred_element_type=jnp.float32)
        u = jax.lax.dot_general(xb, wu, (((1,), (1,)), ((), ())),
                                preferred_element_type=jnp.float32)
        h = (g * jax.lax.logistic(g) * u).astype(jnp.bfloat16)
        y = jax.lax.dot_general(h, wd, (((1,), (1,)), ((), ())),
                                preferred_element_type=jnp.float32)
        ys_ref[...] = y.reshape(ys_ref.shape)


def _combine_body(y1_ref, y2_ref, w1_ref, w2_ref, out_ref):
    bt = y1_ref.shape[1]
    y1 = y1_ref[0].reshape(bt, -1).astype(jnp.float32)
    y2 = y2_ref[0].reshape(bt, -1).astype(jnp.float32)
    out_ref[...] = y1 * w1_ref[...] + y2 * w2_ref[...]


def kernel(hidden_states, W_gate, W_g, W_u, W_d):
    b, s, d = hidden_states.shape
    x = hidden_states.reshape(-1, d)
    T, D = x.shape
    E, F, _ = W_g.shape
    K = 2
    BLK = 256
    NB = T * K // BLK + E - 1
    P = NB * BLK
    SL = D // 128
    A = T * K
    NW = 32
    BPW = A // NW

    logits, comb, x_pack = pl.pallas_call(
        _router_body,
        out_shape=(
            jax.ShapeDtypeStruct((T, E), jnp.float32),
            jax.ShapeDtypeStruct((T, E), jnp.float32),
            jax.ShapeDtypeStruct((T, SL, 128), jnp.float32),
        ),
    )(x, W_gate)

    eidx = jnp.arange(E, dtype=jnp.int32)[None, :]
    w1 = jnp.max(comb, axis=1)
    e1 = jnp.argmax(comb, axis=1).astype(jnp.int32)
    comb2 = jnp.where(eidx == e1[:, None], -1.0, comb)
    w2 = jnp.max(comb2, axis=1)
    e2 = jnp.argmax(comb2, axis=1).astype(jnp.int32)

    flat_e = jnp.concatenate([e1, e2])
    oh = (flat_e[:, None] == eidx).astype(jnp.int32)
    csum = jnp.cumsum(oh, axis=0)
    rank = jnp.take_along_axis(csum, flat_e[:, None], axis=1)[:, 0] - 1
    counts = csum[-1]
    nblk = (counts + BLK - 1) // BLK
    seg_start_blk = jnp.concatenate(
        [jnp.zeros((1,), jnp.int32), jnp.cumsum(nblk)[:-1].astype(jnp.int32)])
    dest = seg_start_blk[flat_e] * BLK + rank
    seg_end_blk = seg_start_blk + nblk
    bidx = jnp.arange(NB, dtype=jnp.int32)
    blk_expert = jnp.minimum(
        jnp.sum((bidx[:, None] >= seg_end_blk[None, :]).astype(jnp.int32),
                axis=1), E - 1).astype(jnp.int32)

    xs = _sc_scatter_rows(x_pack, dest, P)

    be_used = jnp.concatenate([blk_expert, seg_end_blk[-1:]])
    grid_spec = pltpu.PrefetchScalarGridSpec(
        num_scalar_prefetch=1,
        grid=(NB,),
        in_specs=[
            pl.BlockSpec((BLK, SL, 128), lambda i, be: (i, 0, 0)),
            pl.BlockSpec((1, F, D), lambda i, be: (be[i], 0, 0)),
            pl.BlockSpec((1, F, D), lambda i, be: (be[i], 0, 0)),
            pl.BlockSpec((1, D, F), lambda i, be: (be[i], 0, 0)),
        ],
        out_specs=pl.BlockSpec((BLK, SL, 128), lambda i, be: (i, 0, 0)),
    )
    ys = pl.pallas_call(
        _gemm_body,
        grid_spec=grid_spec,
        out_shape=jax.ShapeDtypeStruct((P, SL, 128), jnp.float32),
    )(be_used, xs, W_g, W_u, W_d)

    y12 = _sc_gather_rows(ys, dest)
    y12 = y12.reshape(K, T, SL, 128)

    BT = 256
    out = pl.pallas_call(
        _combine_body,
        grid=(T // BT,),
        in_specs=[
            pl.BlockSpec((1, BT, SL, 128), lambda t: (0, t, 0, 0)),
            pl.BlockSpec((1, BT, SL, 128), lambda t: (1, t, 0, 0)),
            pl.BlockSpec((BT, 1), lambda t: (t, 0)),
            pl.BlockSpec((BT, 1), lambda t: (t, 0)),
        ],
        out_specs=pl.BlockSpec((BT, D), lambda t: (t, 0)),
        out_shape=jax.ShapeDtypeStruct((T, D), jnp.float32),
    )(y12, y12, w1[:, None], w2[:, None])

    return out.reshape(b, s, d), logits.reshape(b, s, E)


# speedup vs baseline: 42.4828x; 1.0026x over previous
import functools

import jax
import jax.numpy as jnp
from jax import lax
from jax.experimental import pallas as pl
from jax.experimental.pallas import tpu as pltpu
from jax.experimental.pallas import tpu_sc as plsc


def _router_body(x_ref, wg_ref, logits_ref, comb_ref, xpack_ref):
    x = x_ref[...]
    wg = wg_ref[...]
    logits = jax.lax.dot_general(
        x, wg, (((1,), (1,)), ((), ())),
        preferred_element_type=jnp.float32,
    )
    logits_ref[...] = logits
    T, E = logits.shape
    m = jnp.max(logits, axis=1, keepdims=True)
    ex = jnp.exp(logits - m)
    p = ex / jnp.sum(ex, axis=1, keepdims=True)
    eidx = jax.lax.broadcasted_iota(jnp.int32, (T, E), 1)
    a1 = jnp.argmax(p, axis=1).astype(jnp.int32)
    m1 = jnp.max(p, axis=1)
    mask1 = eidx == a1[:, None]
    p2 = jnp.where(mask1, -1.0, p)
    a2 = jnp.argmax(p2, axis=1).astype(jnp.int32)
    m2 = jnp.max(p2, axis=1)
    denom = m1 + m2
    w1 = m1 / denom
    w2 = m2 / denom
    mask2 = eidx == a2[:, None]
    comb_ref[...] = (jnp.where(mask1, w1[:, None], 0.0)
                     + jnp.where(mask2, w2[:, None], 0.0))
    xpack_ref[...] = x.reshape(xpack_ref.shape)


def _sc_scatter_body(x_hbm, idx_hbm, o_hbm, idx_v, rows_v, sem, *, chunks):
    nc = 2
    bpw = idx_v.shape[0]
    wid = lax.axis_index("subcore") * nc + lax.axis_index("core")
    t = x_hbm.shape[0]
    for c in range(chunks):
        base = wid * (bpw * chunks) + c * bpw
        src = lax.rem(base, t)
        pltpu.sync_copy(idx_hbm.at[pl.ds(base, bpw)], idx_v)
        pltpu.sync_copy(x_hbm.at[pl.ds(src, bpw)], rows_v)
        pltpu.async_copy(rows_v, o_hbm.at[idx_v], sem).wait()


def _sc_gather_body(ys_hbm, idx_hbm, o_hbm, idx_v, rows_v, sem, *, chunks):
    nc = 2
    bpw = idx_v.shape[0]
    wid = lax.axis_index("subcore") * nc + lax.axis_index("core")
    for c in range(chunks):
        base = wid * (bpw * chunks) + c * bpw
        pltpu.sync_copy(idx_hbm.at[pl.ds(base, bpw)], idx_v)
        pltpu.async_copy(ys_hbm.at[idx_v], rows_v, sem).wait()
        pltpu.sync_copy(rows_v, o_hbm.at[pl.ds(base, bpw)])


_SC_MESH = dict(core_axis_name="core", subcore_axis_name="subcore")
_NW = 32


_CHUNKS = 2


def _sc_scatter_rows(x_pack, dest, p):
    a = dest.shape[0]
    _, sl, lanes = x_pack.shape
    bpw = a // (_NW * _CHUNKS)
    run = pl.kernel(
        functools.partial(_sc_scatter_body, chunks=_CHUNKS),
        out_type=jax.ShapeDtypeStruct((p, sl, lanes), x_pack.dtype),
        mesh=plsc.VectorSubcoreMesh(**_SC_MESH),
        scratch_types=[
            pltpu.VMEM((bpw,), jnp.int32),
            pltpu.VMEM((bpw, sl, lanes), x_pack.dtype),
            pltpu.SemaphoreType.DMA,
        ],
    )
    return run(x_pack, dest)


def _sc_gather_rows(ys, dest):
    a = dest.shape[0]
    _, sl, lanes = ys.shape
    bpw = a // (_NW * _CHUNKS)
    run = pl.kernel(
        functools.partial(_sc_gather_body, chunks=_CHUNKS),
        out_type=jax.ShapeDtypeStruct((a, sl, lanes), ys.dtype),
        mesh=plsc.VectorSubcoreMesh(**_SC_MESH),
        scratch_types=[
            pltpu.VMEM((bpw,), jnp.int32),
            pltpu.VMEM((bpw, sl, lanes), ys.dtype),
            pltpu.SemaphoreType.DMA,
        ],
    )
    return run(ys, dest)


def _gemm_body(be_ref, xs_ref, wg_ref, wu_ref, wd_ref, ys_ref):
    nb = pl.num_programs(0)

    @pl.when(pl.program_id(0) < be_ref[nb])
    def _compute():
        blk = xs_ref.shape[0]
        xb = xs_ref[...].reshape(blk, -1).astype(jnp.bfloat16)
        wg = wg_ref[0].astype(jnp.bfloat16)
        wu = wu_ref[0].astype(jnp.bfloat16)
        wd = wd_ref[0].astype(jnp.bfloat16)
        g = jax.lax.dot_general(xb, wg, (((1,), (1,)), ((), ())),
                                preferred_element_type=jnp.float32)
        u = jax.lax.dot_general(xb, wu, (((1,), (1,)), ((), ())),
                                preferred_element_type=jnp.float32)
        h = (g * jax.lax.logistic(g) * u).astype(jnp.bfloat16)
        y = jax.lax.dot_general(h, wd, (((1,), (1,)), ((), ())),
                                preferred_element_type=jnp.float32)
        ys_ref[...] = y.reshape(ys_ref.shape)


def _combine_body(y1_ref, y2_ref, w1_ref, w2_ref, out_ref):
    bt = y1_ref.shape[1]
    y1 = y1_ref[0].reshape(bt, -1).astype(jnp.float32)
    y2 = y2_ref[0].reshape(bt, -1).astype(jnp.float32)
    out_ref[...] = y1 * w1_ref[...] + y2 * w2_ref[...]


def kernel(hidden_states, W_gate, W_g, W_u, W_d):
    b, s, d = hidden_states.shape
    x = hidden_states.reshape(-1, d)
    T, D = x.shape
    E, F, _ = W_g.shape
    K = 2
    BLK = 256
    NB = T * K // BLK + E - 1
    P = NB * BLK
    SL = D // 128
    A = T * K
    NW = 32
    BPW = A // NW

    logits, comb, x_pack = pl.pallas_call(
        _router_body,
        out_shape=(
            jax.ShapeDtypeStruct((T, E), jnp.float32),
            jax.ShapeDtypeStruct((T, E), jnp.float32),
            jax.ShapeDtypeStruct((T, SL, 128), jnp.float32),
        ),
    )(x, W_gate)

    eidx = jnp.arange(E, dtype=jnp.int32)[None, :]
    w1 = jnp.max(comb, axis=1)
    e1 = jnp.argmax(comb, axis=1).astype(jnp.int32)
    comb2 = jnp.where(eidx == e1[:, None], -1.0, comb)
    w2 = jnp.max(comb2, axis=1)
    e2 = jnp.argmax(comb2, axis=1).astype(jnp.int32)

    flat_e = jnp.concatenate([e1, e2])
    oh = (flat_e[:, None] == eidx).astype(jnp.int32)
    csum = jnp.cumsum(oh, axis=0)
    rank = jnp.take_along_axis(csum, flat_e[:, None], axis=1)[:, 0] - 1
    counts = csum[-1]
    nblk = (counts + BLK - 1) // BLK
    seg_start_blk = jnp.concatenate(
        [jnp.zeros((1,), jnp.int32), jnp.cumsum(nblk)[:-1].astype(jnp.int32)])
    dest = seg_start_blk[flat_e] * BLK + rank
    seg_end_blk = seg_start_blk + nblk
    bidx = jnp.arange(NB, dtype=jnp.int32)
    blk_expert = jnp.minimum(
        jnp.sum((bidx[:, None] >= seg_end_blk[None, :]).astype(jnp.int32),
                axis=1), E - 1).astype(jnp.int32)

    xs = _sc_scatter_rows(x_pack, dest, P)

    be_used = jnp.concatenate([blk_expert, seg_end_blk[-1:]])
    grid_spec = pltpu.PrefetchScalarGridSpec(
        num_scalar_prefetch=1,
        grid=(NB,),
        in_specs=[
            pl.BlockSpec((BLK, SL, 128), lambda i, be: (i, 0, 0)),
            pl.BlockSpec((1, F, D), lambda i, be: (be[i], 0, 0)),
            pl.BlockSpec((1, F, D), lambda i, be: (be[i], 0, 0)),
            pl.BlockSpec((1, D, F), lambda i, be: (be[i], 0, 0)),
        ],
        out_specs=pl.BlockSpec((BLK, SL, 128), lambda i, be: (i, 0, 0)),
    )
    ys = pl.pallas_call(
        _gemm_body,
        grid_spec=grid_spec,
        out_shape=jax.ShapeDtypeStruct((P, SL, 128), jnp.float32),
        compiler_params=pltpu.CompilerParams(
            dimension_semantics=("parallel",)),
    )(be_used, xs, W_g, W_u, W_d)

    y12 = _sc_gather_rows(ys, dest)
    y12 = y12.reshape(K, T, SL, 128)

    BT = 256
    out = pl.pallas_call(
        _combine_body,
        grid=(T // BT,),
        in_specs=[
            pl.BlockSpec((1, BT, SL, 128), lambda t: (0, t, 0, 0)),
            pl.BlockSpec((1, BT, SL, 128), lambda t: (1, t, 0, 0)),
            pl.BlockSpec((BT, 1), lambda t: (t, 0)),
            pl.BlockSpec((BT, 1), lambda t: (t, 0)),
        ],
        out_specs=pl.BlockSpec((BT, D), lambda t: (t, 0)),
        out_shape=jax.ShapeDtypeStruct((T, D), jnp.float32),
    )(y12, y12, w1[:, None], w2[:, None])

    return out.reshape(b, s, d), logits.reshape(b, s, E)
